# Initial kernel scaffold; baseline (speedup 1.0000x reference)
import jax
import jax.numpy as jnp
from jax.experimental import pallas as pl


def kernel(x, pos, normal, batch, W1, b1, W2, b2, W3, b3):
    raise NotImplementedError("write your pallas kernel here")


# speedup vs baseline: 1.2341x; 1.2341x over previous
import functools

import jax
import jax.numpy as jnp
import numpy as np
from jax import lax
from jax.experimental import pallas as pl
from jax.experimental.pallas import tpu as pltpu

N = 10000
D = 128
K = 64
R = 0.25
NB = 8
H = 64
OUT = 128
NP = 10240
BN = 128


def _mm_body(x_ref, w_ref, o_ref):
    o_ref[...] = jnp.dot(x_ref[...], w_ref[...],
                         preferred_element_type=jnp.float32)


def _xw_matmul(x_p, W1a):
    return pl.pallas_call(
        _mm_body,
        grid=(NP // 128,),
        in_specs=[pl.BlockSpec((128, D), lambda i: (i, 0)),
                  pl.BlockSpec((D, H), lambda i: (0, 0))],
        out_specs=pl.BlockSpec((128, H), lambda i: (i, 0)),
        out_shape=jax.ShapeDtypeStruct((NP, H), jnp.float32),
    )(x_p, W1a)


_PAIRS = [
    (0, 0), (1, 1), (2, 2),
    (3, 0), (4, 1), (5, 2),
    (6, 0), (7, 1), (8, 2),
    (3, 6), (4, 7), (5, 8),
    (4, 2), (5, 1),
    (5, 0), (3, 2),
    (3, 1), (4, 0),
    (7, 2), (8, 1),
    (8, 0), (6, 2),
    (6, 1), (7, 0),
    (4, 8), (5, 7),
    (5, 6), (3, 8),
    (3, 7), (4, 6),
]


def _make_consts():
    M1 = np.zeros((8, 16), np.float32)
    M2 = np.zeros((8, 16), np.float32)
    for a in range(3):
        M1[a, a] = 1.0
        M2[a, a] = -1.0
        M1[3 + a, 6 + a] = 1.0
        M2[3 + a, 3 + a] = 1.0
    A = np.zeros((16, 32), np.float32)
    B = np.zeros((16, 32), np.float32)
    for j, (a, b) in enumerate(_PAIRS):
        A[a, j] = 1.0
        B[b, j] = 1.0
    C1 = np.zeros((32, 16), np.float32)
    for j in range(3):
        C1[j, 0] = 1.0
        C1[3 + j, 1] = 1.0
        C1[6 + j, 2] = 1.0
        C1[9 + j, 3] = 1.0
    for c in range(9):
        C1[12 + 2 * c, 4 + c] = 1.0
        C1[13 + 2 * c, 4 + c] = -1.0
    C2 = np.zeros((16, 8), np.float32)
    for k in range(3):
        for c in range(3):
            C2[4 + 3 * k + c, 1 + k] = 1.0
    return (jnp.asarray(M1), jnp.asarray(M2), jnp.asarray(A),
            jnp.asarray(B), jnp.asarray(C1), jnp.asarray(C2))


def _mlp_body(xg_ref, pnj_ref, pni_ref, m1_ref, m2_ref, a_ref, b_ref,
              c1_ref, c2_ref, ws_ref, wc_ref, b1_ref, w2_ref, b2_ref,
              w3_ref, b3_ref, o_ref):
    f32 = jnp.float32

    def mm(a, b):
        return jnp.dot(a, b, preferred_element_type=f32)

    pnj = pnj_ref[...]
    pni = pni_ref[...]
    Z = mm(pnj, m1_ref[...]) + mm(pni, m2_ref[...])
    P1 = mm(Z, a_ref[...]) * mm(Z, b_ref[...])
    P2 = mm(P1, c1_ref[...])
    P3 = P2 * P2
    P2s = P2[:, 0:8]
    lane = lax.broadcasted_iota(jnp.int32, P2s.shape, 1)
    Cv = jnp.where(lane == 0, P2s, mm(P3, c2_ref[...]))
    Nv = jnp.sqrt(Cv)
    H2 = Nv * Nv + P2s * P2s
    Rv = lax.rsqrt(jnp.where(H2 == 0.0, 1.0, H2))
    SINf = jnp.where(lane == 0, Nv * (1.0 / R), Nv * Rv)
    COSv = jnp.where(H2 == 0.0, 1.0, P2s * Rv)
    h1ppf = mm(SINf, ws_ref[...]) + mm(COSv, wc_ref[...])
    h1 = jnp.maximum(xg_ref[...] + h1ppf + b1_ref[...], 0.0)
    h2 = mm(h1, w2_ref[...]) + b2_ref[...]
    hm = jnp.max(h2.reshape(BN, K, H), axis=1)
    o_ref[...] = mm(hm, w3_ref[...]) + b3_ref[...]


def _mlp_call(xg, pnj, pni, Ws8, Wc8, b1, W2, b2, W3, b3):
    nblk = NP // BN
    consts = _make_consts()
    big = lambda i: (i, 0)
    rep = lambda i: (0, 0)
    cshapes = [(8, 16), (8, 16), (16, 32), (16, 32), (32, 16), (16, 8)]
    return pl.pallas_call(
        _mlp_body,
        grid=(nblk,),
        in_specs=[
            pl.BlockSpec((BN * K, H), big),
            pl.BlockSpec((BN * K, 8), big),
            pl.BlockSpec((BN * K, 8), big),
            *[pl.BlockSpec(s, rep) for s in cshapes],
            pl.BlockSpec((8, H), rep),
            pl.BlockSpec((8, H), rep),
            pl.BlockSpec((1, H), rep),
            pl.BlockSpec((H, H), rep),
            pl.BlockSpec((1, H), rep),
            pl.BlockSpec((H, OUT), rep),
            pl.BlockSpec((1, OUT), rep),
        ],
        out_specs=pl.BlockSpec((BN, OUT), big),
        out_shape=jax.ShapeDtypeStruct((NP, OUT), jnp.float32),
    )(xg, pnj, pni, *consts, Ws8, Wc8, b1, W2, b2, W3, b3)


def _nbrs_jnp(pos, batch):
    chunks = []
    step = 2000
    for s in range(0, pos.shape[0], step):
        q = pos[s:s + step]
        d2 = jnp.sum((q[:, None, :] - pos[None, :, :]) ** 2, axis=-1)
        valid = (batch[s:s + step][:, None] == batch[None, :]) & (d2 <= R * R)
        d2m = jnp.where(valid, d2, 1e30)
        vals, idx = lax.top_k(-d2m, K)
        qi = jnp.arange(s, s + q.shape[0], dtype=idx.dtype)[:, None]
        idx = jnp.where(vals <= -1e29, qi, idx)
        chunks.append(idx)
    return jnp.concatenate(chunks, axis=0)


def kernel(x, pos, normal, batch, W1, b1, W2, b2, W3, b3):
    W1a = W1[:D]
    W1b = W1[D:]
    zrow = jnp.zeros((1, H), jnp.float32)
    Ws8 = jnp.concatenate(
        [W1b[0:1], W1b[1:2], W1b[3:4], W1b[5:6], zrow, zrow, zrow, zrow], 0)
    Wc8 = jnp.concatenate(
        [zrow, W1b[2:3], W1b[4:5], W1b[6:7], zrow, zrow, zrow, zrow], 0)
    x_p = jnp.pad(x, ((0, NP - N), (0, 0)))
    xw = _xw_matmul(x_p, W1a)

    nbr = _nbrs_jnp(pos, batch)
    pad_idx = jnp.broadcast_to(
        jnp.arange(N, NP, dtype=nbr.dtype)[:, None], (NP - N, K))
    nbr_p = jnp.concatenate([nbr, pad_idx], axis=0)
    flat = nbr_p.reshape(-1)

    pn = jnp.concatenate(
        [pos, normal, jnp.zeros((N, 2), jnp.float32)], axis=1)
    pn_p = jnp.pad(pn, ((0, NP - N), (0, 0)))

    xg = xw[flat]
    pnj = pn_p[flat]
    pni = jnp.repeat(pn_p, K, axis=0)

    b1r = b1.reshape(1, H)
    b2r = b2.reshape(1, H)
    b3r = b3.reshape(1, OUT)
    out_full = _mlp_call(xg, pnj, pni, Ws8, Wc8, b1r, W2, b2r, W3, b3r)
    return (out_full[:N], pos, batch)


# speedup vs baseline: 12.9762x; 10.5150x over previous
import functools

import jax
import jax.numpy as jnp
import numpy as np
from jax import lax
from jax.experimental import pallas as pl
from jax.experimental.pallas import tpu as pltpu
from jax.experimental.pallas import tpu_sc as plsc

N = 10000
D = 128
K = 64
R = 0.25
NB = 8
H = 64
OUT = 128
NP = 10240
BN = 128


def _mm_body(x_ref, w_ref, o_ref):
    o_ref[...] = jnp.dot(x_ref[...], w_ref[...],
                         preferred_element_type=jnp.float32)


def _xw_matmul(x_p, W1a):
    return pl.pallas_call(
        _mm_body,
        grid=(NP // 128,),
        in_specs=[pl.BlockSpec((128, D), lambda i: (i, 0)),
                  pl.BlockSpec((D, H), lambda i: (0, 0))],
        out_specs=pl.BlockSpec((128, H), lambda i: (i, 0)),
        out_shape=jax.ShapeDtypeStruct((NP, H), jnp.float32),
    )(x_p, W1a)


_PAIRS = [
    (0, 0), (1, 1), (2, 2),
    (3, 0), (4, 1), (5, 2),
    (6, 0), (7, 1), (8, 2),
    (3, 6), (4, 7), (5, 8),
    (4, 2), (5, 1),
    (5, 0), (3, 2),
    (3, 1), (4, 0),
    (7, 2), (8, 1),
    (8, 0), (6, 2),
    (6, 1), (7, 0),
    (4, 8), (5, 7),
    (5, 6), (3, 8),
    (3, 7), (4, 6),
]


def _make_consts():
    M1 = np.zeros((16, 16), np.float32)
    M2 = np.zeros((8, 16), np.float32)
    for a in range(3):
        M1[a, a] = 1.0
        M2[a, a] = -1.0
        M1[3 + a, 6 + a] = 1.0
        M2[3 + a, 3 + a] = 1.0
    A = np.zeros((16, 32), np.float32)
    B = np.zeros((16, 32), np.float32)
    for j, (a, b) in enumerate(_PAIRS):
        A[a, j] = 1.0
        B[b, j] = 1.0
    C1 = np.zeros((32, 16), np.float32)
    for j in range(3):
        C1[j, 0] = 1.0
        C1[3 + j, 1] = 1.0
        C1[6 + j, 2] = 1.0
        C1[9 + j, 3] = 1.0
    for c in range(9):
        C1[12 + 2 * c, 4 + c] = 1.0
        C1[13 + 2 * c, 4 + c] = -1.0
    C2 = np.zeros((16, 8), np.float32)
    for k in range(3):
        for c in range(3):
            C2[4 + 3 * k + c, 1 + k] = 1.0
    return (jnp.asarray(M1), jnp.asarray(M2), jnp.asarray(A),
            jnp.asarray(B), jnp.asarray(C1), jnp.asarray(C2))


def _mlp_body(xg_ref, pnj_ref, pni_ref, m1_ref, m2_ref, a_ref, b_ref,
              c1_ref, c2_ref, ws_ref, wc_ref, b1_ref, w2_ref, b2_ref,
              w3_ref, b3_ref, o_ref):
    f32 = jnp.float32

    def mm(a, b):
        return jnp.dot(a, b, preferred_element_type=f32)

    pnj = pnj_ref[...]
    pni = pni_ref[...]
    Z = mm(pnj, m1_ref[...]) + mm(pni, m2_ref[...])
    P1 = mm(Z, a_ref[...]) * mm(Z, b_ref[...])
    P2 = mm(P1, c1_ref[...])
    P3 = P2 * P2
    P2s = P2[:, 0:8]
    lane = lax.broadcasted_iota(jnp.int32, P2s.shape, 1)
    Cv = jnp.where(lane == 0, P2s, mm(P3, c2_ref[...]))
    Nv = jnp.sqrt(Cv)
    H2 = Nv * Nv + P2s * P2s
    Rv = lax.rsqrt(jnp.where(H2 == 0.0, 1.0, H2))
    SINf = jnp.where(lane == 0, Nv * (1.0 / R), Nv * Rv)
    COSv = jnp.where(H2 == 0.0, 1.0, P2s * Rv)
    h1ppf = mm(SINf, ws_ref[...]) + mm(COSv, wc_ref[...])
    h1 = jnp.maximum(xg_ref[...] + h1ppf + b1_ref[...], 0.0)
    h2 = mm(h1, w2_ref[...]) + b2_ref[...]
    hm = jnp.max(h2.reshape(BN, K, H), axis=1)
    o_ref[...] = mm(hm, w3_ref[...]) + b3_ref[...]


def _mlp_call(xg, pnj, pni, Ws8, Wc8, b1, W2, b2, W3, b3):
    nblk = NP // BN
    consts = _make_consts()
    big = lambda i: (i, 0)
    rep = lambda i: (0, 0)
    cshapes = [(16, 16), (8, 16), (16, 32), (16, 32), (32, 16), (16, 8)]
    return pl.pallas_call(
        _mlp_body,
        grid=(nblk,),
        in_specs=[
            pl.BlockSpec((BN * K, H), big),
            pl.BlockSpec((BN * K, 16), big),
            pl.BlockSpec((BN * K, 8), big),
            *[pl.BlockSpec(s, rep) for s in cshapes],
            pl.BlockSpec((8, H), rep),
            pl.BlockSpec((8, H), rep),
            pl.BlockSpec((1, H), rep),
            pl.BlockSpec((H, H), rep),
            pl.BlockSpec((1, H), rep),
            pl.BlockSpec((H, OUT), rep),
            pl.BlockSpec((1, OUT), rep),
        ],
        out_specs=pl.BlockSpec((BN, OUT), big),
        out_shape=jax.ShapeDtypeStruct((NP, OUT), jnp.float32),
    )(xg, pnj, pni, *consts, Ws8, Wc8, b1, W2, b2, W3, b3)


_NW = 32
_NQW = NP // _NW
_CM = 256
_CROW = _CM + 16
_R2BITS = np.float32(R * R).view(np.int32).item()
_SV = 1


def _stage(px_h, py_h, pz_h, gs_h, ge_h, pxv, pyv, pzv, gsv, gev, qbase):
    pltpu.sync_copy(px_h, pxv.at[pl.ds(0, NP)])
    pltpu.sync_copy(py_h, pyv.at[pl.ds(0, NP)])
    pltpu.sync_copy(pz_h, pzv.at[pl.ds(0, NP)])
    pltpu.sync_copy(gs_h.at[pl.ds(qbase, _NQW)], gsv.at[pl.ds(0, _NQW)])
    pltpu.sync_copy(ge_h.at[pl.ds(qbase, _NQW)], gev.at[pl.ds(0, _NQW)])


_GMAX = 2048


def _scan_body(px_h, py_h, pz_h, gs_h, ge_h, du_h,
               pxv, pyv, pzv, gsv, gev, rowb):
    i32 = jnp.int32
    wid = lax.axis_index("s") * 2 + lax.axis_index("c")
    qbase = wid * _NQW
    _stage(px_h, py_h, pz_h, gs_h, ge_h, pxv, pyv, pzv, gsv, gev, qbase)

    def per_query(qi, _):
        q = qbase + qi
        s = gsv[pl.ds(qi, 16)][0]
        e = gev[pl.ds(qi, 16)][0]
        qx = pxv[pl.ds(q, 16)][0]
        qy = pyv[pl.ds(q, 16)][0]
        qz = pzv[pl.ds(q, 16)][0]
        nchunks = (e - s + 15) // 16

        def scan_chunk(ci, _):
            base = s + ci * 16
            dx = pxv[pl.ds(base, 16)] - qx
            dy = pyv[pl.ds(base, 16)] - qy
            dz = pzv[pl.ds(base, 16)] - qz
            d2 = (dx * dx + dy * dy) + dz * dz
            rowb[pl.ds(ci * 16, 16)] = d2
            return 0

        lax.fori_loop(0, nchunks, scan_chunk, 0)
        pltpu.sync_copy(rowb.at[pl.ds(0, _GMAX)],
                        du_h.at[pl.ds(q * _GMAX, _GMAX)])
        return 0

    lax.fori_loop(0, _NQW, per_query, 0)


def _emitg_body(du_ref, gl_ref, gsx_ref, tg_ref, o_ref):
    i32 = jnp.int32
    f32 = jnp.float32
    bits = lax.bitcast_convert_type(du_ref[...], i32)
    glen = gl_ref[...]
    lanei = lax.broadcasted_iota(i32, (128, _GMAX), 1)
    valid = lanei < glen
    bits = jnp.where(valid, bits, jnp.int32(2147483647))

    def bs_body(_, lh):
        lo, hi = lh
        mid = lo + (hi - lo) // 2
        c = jnp.sum((bits <= mid).astype(i32), axis=1, keepdims=True)
        big = c >= K
        return jnp.where(big, lo, mid), jnp.where(big, mid, hi)

    lo0 = jnp.full((128, 1), -1, i32)
    hi0 = jnp.full((128, 1), _R2BITS, i32)
    _, tstar = lax.fori_loop(0, 31, bs_body, (lo0, hi0))

    sel = (bits <= tstar).astype(f32)
    rank = jnp.dot(sel, tg_ref[...],
                   preferred_element_type=f32)
    cnt = jnp.minimum(jnp.sum(sel, axis=1, keepdims=True), jnp.float32(K))
    gsx = gsx_ref[...].astype(f32)
    idxval = (gsx + lanei.astype(f32)) * sel
    cols = []
    for j in range(K):
        pick = jnp.where(rank == jnp.float32(j + 1), idxval, 0.0)
        cols.append(jnp.sum(pick, axis=1, keepdims=True))
    V = jnp.concatenate(cols, axis=1)
    i = pl.program_id(0)
    qf = (i * 128 + lax.broadcasted_iota(i32, (128, K), 0)).astype(f32)
    colj = lax.broadcasted_iota(i32, (128, K), 1).astype(f32)
    o_ref[...] = jnp.where(colj < cnt, V, qf).astype(i32)


def _emit_call(du, glen, gsx):
    tg = jnp.asarray(np.triu(np.ones((_GMAX, _GMAX), np.float32)))
    return pl.pallas_call(
        _emitg_body,
        grid=(NP // 128,),
        in_specs=[pl.BlockSpec((128, _GMAX), lambda i: (i, 0)),
                  pl.BlockSpec((128, 1), lambda i: (i, 0)),
                  pl.BlockSpec((128, 1), lambda i: (i, 0)),
                  pl.BlockSpec((_GMAX, _GMAX), lambda i: (0, 0))],
        out_specs=pl.BlockSpec((128, K), lambda i: (i, 0)),
        out_shape=jax.ShapeDtypeStruct((NP, K), jnp.int32),
    )(du, glen, gsx, tg)


_GC = 64


def _gather_body(tbl_h, pn_h, idx_h, xg_h, pnj_h, idxv, rowsv, pnv, sem):
    cid = lax.axis_index("c")
    sid = lax.axis_index("s")
    wid = sid * 2 + cid
    nrows = (NP * K) // _GC // _NW
    rbase = wid * nrows

    def step(ci, _):
        row = rbase + ci
        pltpu.sync_copy(idx_h.at[row], idxv)
        pltpu.async_copy(tbl_h.at[idxv], rowsv, sem).wait()
        pltpu.async_copy(pn_h.at[idxv], pnv, sem).wait()
        pltpu.sync_copy(rowsv, xg_h.at[pl.ds(row * _GC, _GC)])
        pltpu.sync_copy(pnv, pnj_h.at[pl.ds(row * _GC, _GC)])
        return 0

    lax.fori_loop(0, nrows, step, 0)


def _sc_gather(tbl, pn, idx2d):
    mesh = plsc.VectorSubcoreMesh(core_axis_name="c", subcore_axis_name="s")
    f = pl.kernel(
        _gather_body, mesh=mesh,
        compiler_params=pltpu.CompilerParams(use_tc_tiling_on_sc=False),
        out_type=(jax.ShapeDtypeStruct((NP * K, H), jnp.float32),
                  jax.ShapeDtypeStruct((NP * K, 16), jnp.float32)),
        scratch_types=[
            pltpu.VMEM((_GC,), jnp.int32),
            pltpu.VMEM((_GC, H), jnp.float32),
            pltpu.VMEM((_GC, 16), jnp.float32),
            pltpu.SemaphoreType.DMA,
        ])
    return f(tbl, pn, idx2d)


def _sc_search_gather(px, py, pz, gs, ge, tbl, pn):
    mesh = plsc.VectorSubcoreMesh(core_axis_name="c", subcore_axis_name="s")
    stage_scratch = [
        pltpu.VMEM((NP + 16,), jnp.float32),
        pltpu.VMEM((NP + 16,), jnp.float32),
        pltpu.VMEM((NP + 16,), jnp.float32),
        pltpu.VMEM((_NQW + 16,), jnp.int32),
        pltpu.VMEM((_NQW + 16,), jnp.int32),
    ]
    scan = pl.kernel(
        _scan_body, mesh=mesh,
        out_type=jax.ShapeDtypeStruct((NP * _GMAX,), jnp.float32),
        scratch_types=stage_scratch + [
            pltpu.VMEM((_GMAX + 16,), jnp.float32),
        ])
    du = scan(px, py, pz, gs, ge)
    glen = (ge - gs).reshape(NP, 1)
    idx2d = _emit_call(du.reshape(NP, _GMAX), glen, gs.reshape(NP, 1))
    return _sc_gather(tbl, pn, idx2d)


def _nbrs_jnp(pos, batch):
    chunks = []
    step = 2000
    for s in range(0, pos.shape[0], step):
        q = pos[s:s + step]
        d2 = jnp.sum((q[:, None, :] - pos[None, :, :]) ** 2, axis=-1)
        valid = (batch[s:s + step][:, None] == batch[None, :]) & (d2 <= R * R)
        d2m = jnp.where(valid, d2, 1e30)
        vals, idx = lax.top_k(-d2m, K)
        qi = jnp.arange(s, s + q.shape[0], dtype=idx.dtype)[:, None]
        idx = jnp.where(vals <= -1e29, qi, idx)
        chunks.append(idx)
    return jnp.concatenate(chunks, axis=0)


def kernel(x, pos, normal, batch, W1, b1, W2, b2, W3, b3):
    W1a = W1[:D]
    W1b = W1[D:]
    zrow = jnp.zeros((1, H), jnp.float32)
    Ws8 = jnp.concatenate(
        [W1b[0:1], W1b[1:2], W1b[3:4], W1b[5:6], zrow, zrow, zrow, zrow], 0)
    Wc8 = jnp.concatenate(
        [zrow, W1b[2:3], W1b[4:5], W1b[6:7], zrow, zrow, zrow, zrow], 0)
    x_p = jnp.pad(x, ((0, NP - N), (0, 0)))
    xw = _xw_matmul(x_p, W1a)

    pos_p = jnp.pad(pos, ((0, NP - N), (0, 0)))
    px, py, pz = pos_p[:, 0], pos_p[:, 1], pos_p[:, 2]
    starts = jnp.searchsorted(batch, jnp.arange(NB, dtype=batch.dtype),
                              side="left").astype(jnp.int32)
    ends = jnp.searchsorted(batch, jnp.arange(NB, dtype=batch.dtype),
                            side="right").astype(jnp.int32)
    gs = jnp.pad(starts[batch], (0, NP - N))
    ge = jnp.pad(ends[batch], (0, NP - N))
    pn = jnp.concatenate(
        [pos, normal, jnp.zeros((N, 10), jnp.float32)], axis=1)
    pn_p = jnp.pad(pn, ((0, NP - N), (0, 0)))

    xg, pnj = _sc_search_gather(px, py, pz, gs, ge, xw, pn_p)
    pni = jnp.repeat(pn_p[:, :8], K, axis=0)

    b1r = b1.reshape(1, H)
    b2r = b2.reshape(1, H)
    b3r = b3.reshape(1, OUT)
    out_full = _mlp_call(xg, pnj, pni, Ws8, Wc8, b1r, W2, b2r, W3, b3r)
    return (out_full[:N], pos, batch)


# speedup vs baseline: 15.7384x; 1.2129x over previous
import functools

import jax
import jax.numpy as jnp
import numpy as np
from jax import lax
from jax.experimental import pallas as pl
from jax.experimental.pallas import tpu as pltpu
from jax.experimental.pallas import tpu_sc as plsc

N = 10000
D = 128
K = 64
R = 0.25
NB = 8
H = 64
OUT = 128
NP = 10240
BN = 128


def _mm_body(x_ref, w_ref, o_ref):
    o_ref[...] = jnp.dot(x_ref[...], w_ref[...],
                         preferred_element_type=jnp.float32)


def _xw_matmul(x_p, W1a):
    return pl.pallas_call(
        _mm_body,
        grid=(NP // 128,),
        in_specs=[pl.BlockSpec((128, D), lambda i: (i, 0)),
                  pl.BlockSpec((D, H), lambda i: (0, 0))],
        out_specs=pl.BlockSpec((128, H), lambda i: (i, 0)),
        out_shape=jax.ShapeDtypeStruct((NP, H), jnp.float32),
    )(x_p, W1a)


_PAIRS = [
    (0, 0), (1, 1), (2, 2),
    (3, 0), (4, 1), (5, 2),
    (6, 0), (7, 1), (8, 2),
    (3, 6), (4, 7), (5, 8),
    (4, 2), (5, 1),
    (5, 0), (3, 2),
    (3, 1), (4, 0),
    (7, 2), (8, 1),
    (8, 0), (6, 2),
    (6, 1), (7, 0),
    (4, 8), (5, 7),
    (5, 6), (3, 8),
    (3, 7), (4, 6),
]


def _make_consts():
    M1 = np.zeros((16, 16), np.float32)
    M2 = np.zeros((8, 16), np.float32)
    for a in range(3):
        M1[a, a] = 1.0
        M2[a, a] = -1.0
        M1[3 + a, 6 + a] = 1.0
        M2[3 + a, 3 + a] = 1.0
    A = np.zeros((16, 32), np.float32)
    B = np.zeros((16, 32), np.float32)
    for j, (a, b) in enumerate(_PAIRS):
        A[a, j] = 1.0
        B[b, j] = 1.0
    C1 = np.zeros((32, 16), np.float32)
    for j in range(3):
        C1[j, 0] = 1.0
        C1[3 + j, 1] = 1.0
        C1[6 + j, 2] = 1.0
        C1[9 + j, 3] = 1.0
    for c in range(9):
        C1[12 + 2 * c, 4 + c] = 1.0
        C1[13 + 2 * c, 4 + c] = -1.0
    C2 = np.zeros((16, 8), np.float32)
    for k in range(3):
        for c in range(3):
            C2[4 + 3 * k + c, 1 + k] = 1.0
    return (jnp.asarray(M1), jnp.asarray(M2), jnp.asarray(A),
            jnp.asarray(B), jnp.asarray(C1), jnp.asarray(C2))


def _mlp_body(xg_ref, pnj_ref, pni_ref, m1_ref, m2_ref, a_ref, b_ref,
              c1_ref, c2_ref, ws_ref, wc_ref, b1_ref, w2_ref, b2_ref,
              w3_ref, b3_ref, o_ref):
    f32 = jnp.float32

    def mm(a, b):
        return jnp.dot(a, b, preferred_element_type=f32)

    pnj = pnj_ref[...]
    pni = pni_ref[...]
    Z = mm(pnj, m1_ref[...]) + mm(pni, m2_ref[...])
    P1 = mm(Z, a_ref[...]) * mm(Z, b_ref[...])
    P2 = mm(P1, c1_ref[...])
    P3 = P2 * P2
    P2s = P2[:, 0:8]
    lane = lax.broadcasted_iota(jnp.int32, P2s.shape, 1)
    Cv = jnp.where(lane == 0, P2s, mm(P3, c2_ref[...]))
    Nv = jnp.sqrt(Cv)
    H2 = Nv * Nv + P2s * P2s
    Rv = lax.rsqrt(jnp.where(H2 == 0.0, 1.0, H2))
    SINf = jnp.where(lane == 0, Nv * (1.0 / R), Nv * Rv)
    COSv = jnp.where(H2 == 0.0, 1.0, P2s * Rv)
    h1ppf = mm(SINf, ws_ref[...]) + mm(COSv, wc_ref[...])
    h1 = jnp.maximum(xg_ref[...] + h1ppf + b1_ref[...], 0.0)
    h2 = mm(h1, w2_ref[...]) + b2_ref[...]
    hm = jnp.max(h2.reshape(BN, K, H), axis=1)
    o_ref[...] = mm(hm, w3_ref[...]) + b3_ref[...]


def _mlp_call(xg, pnj, pni, Ws8, Wc8, b1, W2, b2, W3, b3):
    nblk = NP // BN
    consts = _make_consts()
    big = lambda i: (i, 0)
    rep = lambda i: (0, 0)
    cshapes = [(16, 16), (8, 16), (16, 32), (16, 32), (32, 16), (16, 8)]
    return pl.pallas_call(
        _mlp_body,
        grid=(nblk,),
        in_specs=[
            pl.BlockSpec((BN * K, H), big),
            pl.BlockSpec((BN * K, 16), big),
            pl.BlockSpec((BN * K, 8), big),
            *[pl.BlockSpec(s, rep) for s in cshapes],
            pl.BlockSpec((8, H), rep),
            pl.BlockSpec((8, H), rep),
            pl.BlockSpec((1, H), rep),
            pl.BlockSpec((H, H), rep),
            pl.BlockSpec((1, H), rep),
            pl.BlockSpec((H, OUT), rep),
            pl.BlockSpec((1, OUT), rep),
        ],
        out_specs=pl.BlockSpec((BN, OUT), big),
        out_shape=jax.ShapeDtypeStruct((NP, OUT), jnp.float32),
    )(xg, pnj, pni, *consts, Ws8, Wc8, b1, W2, b2, W3, b3)


_NW = 32
_NQW = NP // _NW
_CM = 256
_CROW = _CM + 16
_R2BITS = np.float32(R * R).view(np.int32).item()
_SV = 1


def _stage(px_h, py_h, pz_h, gs_h, ge_h, pxv, pyv, pzv, gsv, gev, qbase):
    pltpu.sync_copy(px_h, pxv.at[pl.ds(0, NP)])
    pltpu.sync_copy(py_h, pyv.at[pl.ds(0, NP)])
    pltpu.sync_copy(pz_h, pzv.at[pl.ds(0, NP)])
    pltpu.sync_copy(gs_h.at[pl.ds(qbase, _NQW)], gsv.at[pl.ds(0, _NQW)])
    pltpu.sync_copy(ge_h.at[pl.ds(qbase, _NQW)], gev.at[pl.ds(0, _NQW)])


_GMAX = 2048


def _scan_body(px_h, py_h, pz_h, gs_h, ge_h, du_h,
               pxv, pyv, pzv, gsv, gev, rowb):
    i32 = jnp.int32
    wid = lax.axis_index("s") * 2 + lax.axis_index("c")
    qbase = wid * _NQW
    _stage(px_h, py_h, pz_h, gs_h, ge_h, pxv, pyv, pzv, gsv, gev, qbase)

    def per_query(qi, _):
        q = qbase + qi
        s = gsv[pl.ds(qi, 16)][0]
        e = gev[pl.ds(qi, 16)][0]
        qx = pxv[pl.ds(q, 16)][0]
        qy = pyv[pl.ds(q, 16)][0]
        qz = pzv[pl.ds(q, 16)][0]
        nchunks = (e - s + 15) // 16

        def scan_chunk(ci, _):
            base = s + ci * 16
            dx = pxv[pl.ds(base, 16)] - qx
            dy = pyv[pl.ds(base, 16)] - qy
            dz = pzv[pl.ds(base, 16)] - qz
            d2 = (dx * dx + dy * dy) + dz * dz
            rowb[pl.ds(ci * 16, 16)] = d2
            return 0

        lax.fori_loop(0, nchunks, scan_chunk, 0)
        pltpu.sync_copy(rowb.at[pl.ds(0, _GMAX)],
                        du_h.at[pl.ds(q * _GMAX, _GMAX)])
        return 0

    lax.fori_loop(0, _NQW, per_query, 0)


def _emitg_body(du_ref, gl_ref, gsx_ref, tg_ref, o_ref):
    i32 = jnp.int32
    f32 = jnp.float32
    bits = lax.bitcast_convert_type(du_ref[...], i32)
    glen = gl_ref[...]
    lanei = lax.broadcasted_iota(i32, (128, _GMAX), 1)
    valid = lanei < glen
    bits = jnp.where(valid, bits, jnp.int32(2147483647))

    def bs_body(_, lh):
        lo, hi = lh
        mid = lo + (hi - lo) // 2
        c = jnp.sum((bits <= mid).astype(i32), axis=1, keepdims=True)
        big = c >= K
        return jnp.where(big, lo, mid), jnp.where(big, mid, hi)

    lo0 = jnp.full((128, 1), -1, i32)
    hi0 = jnp.full((128, 1), _R2BITS, i32)
    _, tstar = lax.fori_loop(0, 31, bs_body, (lo0, hi0))

    sel = (bits <= tstar).astype(f32)
    rank = jnp.dot(sel, tg_ref[...],
                   preferred_element_type=f32)
    cnt = jnp.minimum(jnp.sum(sel, axis=1, keepdims=True), jnp.float32(K))
    gsx = gsx_ref[...].astype(f32)
    idxval = (gsx + lanei.astype(f32)) * sel
    cols = []
    for j in range(K):
        pick = jnp.where(rank == jnp.float32(j + 1), idxval, 0.0)
        cols.append(jnp.sum(pick, axis=1, keepdims=True))
    V = jnp.concatenate(cols, axis=1)
    i = pl.program_id(0)
    qf = (i * 128 + lax.broadcasted_iota(i32, (128, K), 0)).astype(f32)
    colj = lax.broadcasted_iota(i32, (128, K), 1).astype(f32)
    o_ref[...] = jnp.where(colj < cnt, V, qf).astype(i32)


def _emit_call(du, glen, gsx):
    tg = jnp.asarray(np.triu(np.ones((_GMAX, _GMAX), np.float32)))
    return pl.pallas_call(
        _emitg_body,
        grid=(NP // 128,),
        in_specs=[pl.BlockSpec((128, _GMAX), lambda i: (i, 0)),
                  pl.BlockSpec((128, 1), lambda i: (i, 0)),
                  pl.BlockSpec((128, 1), lambda i: (i, 0)),
                  pl.BlockSpec((_GMAX, _GMAX), lambda i: (0, 0))],
        out_specs=pl.BlockSpec((128, K), lambda i: (i, 0)),
        out_shape=jax.ShapeDtypeStruct((NP, K), jnp.int32),
    )(du, glen, gsx, tg)


_GC = 512


def _gather_body(tbl_h, pn_h, idx_h, xg_h, pnj_h, idxv, rowsv, pnv, sem):
    cid = lax.axis_index("c")
    sid = lax.axis_index("s")
    wid = sid * 2 + cid
    nw = (NP * K) // _NW
    base = wid * nw

    def step(ci, _):
        off = base + ci * _GC
        pltpu.sync_copy(idx_h.at[pl.ds(off, _GC)], idxv)
        cp1 = pltpu.async_copy(tbl_h.at[idxv], rowsv, sem)
        cp2 = pltpu.async_copy(pn_h.at[idxv], pnv, sem)
        cp1.wait()
        cp2.wait()
        pltpu.sync_copy(rowsv, xg_h.at[pl.ds(off, _GC)])
        pltpu.sync_copy(pnv, pnj_h.at[pl.ds(off, _GC)])
        return 0

    lax.fori_loop(0, nw // _GC, step, 0)


def _sc_gather(tbl, pn, idx2d):
    mesh = plsc.VectorSubcoreMesh(core_axis_name="c", subcore_axis_name="s")
    f = pl.kernel(
        _gather_body, mesh=mesh,
        compiler_params=pltpu.CompilerParams(use_tc_tiling_on_sc=False),
        out_type=(jax.ShapeDtypeStruct((NP * K, H), jnp.float32),
                  jax.ShapeDtypeStruct((NP * K, 16), jnp.float32)),
        scratch_types=[
            pltpu.VMEM((_GC,), jnp.int32),
            pltpu.VMEM((_GC, H), jnp.float32),
            pltpu.VMEM((_GC, 16), jnp.float32),
            pltpu.SemaphoreType.DMA,
        ])
    return f(tbl, pn, idx2d)


def _sc_search_gather(px, py, pz, gs, ge, tbl, pn):
    mesh = plsc.VectorSubcoreMesh(core_axis_name="c", subcore_axis_name="s")
    stage_scratch = [
        pltpu.VMEM((NP + 16,), jnp.float32),
        pltpu.VMEM((NP + 16,), jnp.float32),
        pltpu.VMEM((NP + 16,), jnp.float32),
        pltpu.VMEM((_NQW + 16,), jnp.int32),
        pltpu.VMEM((_NQW + 16,), jnp.int32),
    ]
    scan = pl.kernel(
        _scan_body, mesh=mesh,
        out_type=jax.ShapeDtypeStruct((NP * _GMAX,), jnp.float32),
        scratch_types=stage_scratch + [
            pltpu.VMEM((_GMAX + 16,), jnp.float32),
        ])
    du = scan(px, py, pz, gs, ge)
    glen = (ge - gs).reshape(NP, 1)
    idx2d = _emit_call(du.reshape(NP, _GMAX), glen, gs.reshape(NP, 1))
    return _sc_gather(tbl, pn, idx2d.reshape(NP * K))


def _nbrs_jnp(pos, batch):
    chunks = []
    step = 2000
    for s in range(0, pos.shape[0], step):
        q = pos[s:s + step]
        d2 = jnp.sum((q[:, None, :] - pos[None, :, :]) ** 2, axis=-1)
        valid = (batch[s:s + step][:, None] == batch[None, :]) & (d2 <= R * R)
        d2m = jnp.where(valid, d2, 1e30)
        vals, idx = lax.top_k(-d2m, K)
        qi = jnp.arange(s, s + q.shape[0], dtype=idx.dtype)[:, None]
        idx = jnp.where(vals <= -1e29, qi, idx)
        chunks.append(idx)
    return jnp.concatenate(chunks, axis=0)


def kernel(x, pos, normal, batch, W1, b1, W2, b2, W3, b3):
    W1a = W1[:D]
    W1b = W1[D:]
    zrow = jnp.zeros((1, H), jnp.float32)
    Ws8 = jnp.concatenate(
        [W1b[0:1], W1b[1:2], W1b[3:4], W1b[5:6], zrow, zrow, zrow, zrow], 0)
    Wc8 = jnp.concatenate(
        [zrow, W1b[2:3], W1b[4:5], W1b[6:7], zrow, zrow, zrow, zrow], 0)
    x_p = jnp.pad(x, ((0, NP - N), (0, 0)))
    xw = _xw_matmul(x_p, W1a)

    pos_p = jnp.pad(pos, ((0, NP - N), (0, 0)))
    px, py, pz = pos_p[:, 0], pos_p[:, 1], pos_p[:, 2]
    starts = jnp.searchsorted(batch, jnp.arange(NB, dtype=batch.dtype),
                              side="left").astype(jnp.int32)
    ends = jnp.searchsorted(batch, jnp.arange(NB, dtype=batch.dtype),
                            side="right").astype(jnp.int32)
    gs = jnp.pad(starts[batch], (0, NP - N))
    ge = jnp.pad(ends[batch], (0, NP - N))
    pn = jnp.concatenate(
        [pos, normal, jnp.zeros((N, 10), jnp.float32)], axis=1)
    pn_p = jnp.pad(pn, ((0, NP - N), (0, 0)))

    xg, pnj = _sc_search_gather(px, py, pz, gs, ge, xw, pn_p)
    pni = jnp.repeat(pn_p[:, :8], K, axis=0)

    b1r = b1.reshape(1, H)
    b2r = b2.reshape(1, H)
    b3r = b3.reshape(1, OUT)
    out_full = _mlp_call(xg, pnj, pni, Ws8, Wc8, b1r, W2, b2r, W3, b3r)
    return (out_full[:N], pos, batch)


# speedup vs baseline: 15.9514x; 1.0135x over previous
import functools

import jax
import jax.numpy as jnp
import numpy as np
from jax import lax
from jax.experimental import pallas as pl
from jax.experimental.pallas import tpu as pltpu
from jax.experimental.pallas import tpu_sc as plsc

N = 10000
D = 128
K = 64
R = 0.25
NB = 8
H = 64
OUT = 128
NP = 10240
BN = 128


def _mm_body(x_ref, w_ref, o_ref):
    o_ref[...] = jnp.dot(x_ref[...], w_ref[...],
                         preferred_element_type=jnp.float32)


def _xw_matmul(x_p, W1a):
    return pl.pallas_call(
        _mm_body,
        grid=(NP // 128,),
        in_specs=[pl.BlockSpec((128, D), lambda i: (i, 0)),
                  pl.BlockSpec((D, H), lambda i: (0, 0))],
        out_specs=pl.BlockSpec((128, H), lambda i: (i, 0)),
        out_shape=jax.ShapeDtypeStruct((NP, H), jnp.float32),
    )(x_p, W1a)


_PAIRS = [
    (0, 0), (1, 1), (2, 2),
    (3, 0), (4, 1), (5, 2),
    (6, 0), (7, 1), (8, 2),
    (3, 6), (4, 7), (5, 8),
    (4, 2), (5, 1),
    (5, 0), (3, 2),
    (3, 1), (4, 0),
    (7, 2), (8, 1),
    (8, 0), (6, 2),
    (6, 1), (7, 0),
    (4, 8), (5, 7),
    (5, 6), (3, 8),
    (3, 7), (4, 6),
]


def _make_consts():
    M1 = np.zeros((16, 16), np.float32)
    M2 = np.zeros((8, 16), np.float32)
    for a in range(3):
        M1[a, a] = 1.0
        M2[a, a] = -1.0
        M1[3 + a, 6 + a] = 1.0
        M2[3 + a, 3 + a] = 1.0
    A = np.zeros((16, 32), np.float32)
    B = np.zeros((16, 32), np.float32)
    for j, (a, b) in enumerate(_PAIRS):
        A[a, j] = 1.0
        B[b, j] = 1.0
    C1 = np.zeros((32, 16), np.float32)
    for j in range(3):
        C1[j, 0] = 1.0
        C1[3 + j, 1] = 1.0
        C1[6 + j, 2] = 1.0
        C1[9 + j, 3] = 1.0
    for c in range(9):
        C1[12 + 2 * c, 4 + c] = 1.0
        C1[13 + 2 * c, 4 + c] = -1.0
    C2 = np.zeros((16, 8), np.float32)
    for k in range(3):
        for c in range(3):
            C2[4 + 3 * k + c, 1 + k] = 1.0
    return (jnp.asarray(M1), jnp.asarray(M2), jnp.asarray(A),
            jnp.asarray(B), jnp.asarray(C1), jnp.asarray(C2))


def _mlp_body(xg_ref, pnj_ref, pni_ref, m1_ref, m2_ref, a_ref, b_ref,
              c1_ref, c2_ref, ws_ref, wc_ref, b1_ref, w2_ref, b2_ref,
              w3_ref, b3_ref, o_ref):
    f32 = jnp.float32

    def mm(a, b):
        return jnp.dot(a, b, preferred_element_type=f32)

    pnj = pnj_ref[...]
    pni = pni_ref[...]
    Z = mm(pnj, m1_ref[...]) + mm(pni, m2_ref[...])
    P1 = mm(Z, a_ref[...]) * mm(Z, b_ref[...])
    P2 = mm(P1, c1_ref[...])
    P3 = P2 * P2
    P2s = P2[:, 0:8]
    lane = lax.broadcasted_iota(jnp.int32, P2s.shape, 1)
    Cv = jnp.where(lane == 0, P2s, mm(P3, c2_ref[...]))
    Nv = jnp.sqrt(Cv)
    H2 = Nv * Nv + P2s * P2s
    Rv = lax.rsqrt(jnp.where(H2 == 0.0, 1.0, H2))
    SINf = jnp.where(lane == 0, Nv * (1.0 / R), Nv * Rv)
    COSv = jnp.where(H2 == 0.0, 1.0, P2s * Rv)
    h1ppf = mm(SINf, ws_ref[...]) + mm(COSv, wc_ref[...])
    h1 = jnp.maximum(xg_ref[...] + h1ppf + b1_ref[...], 0.0)
    h2 = mm(h1, w2_ref[...]) + b2_ref[...]
    hm = jnp.max(h2.reshape(BN, K, H), axis=1)
    o_ref[...] = mm(hm, w3_ref[...]) + b3_ref[...]


def _mlp_call(xg, pnj, pni, Ws8, Wc8, b1, W2, b2, W3, b3):
    nblk = NP // BN
    consts = _make_consts()
    big = lambda i: (i, 0)
    rep = lambda i: (0, 0)
    cshapes = [(16, 16), (8, 16), (16, 32), (16, 32), (32, 16), (16, 8)]
    return pl.pallas_call(
        _mlp_body,
        grid=(nblk,),
        in_specs=[
            pl.BlockSpec((BN * K, H), big),
            pl.BlockSpec((BN * K, 16), big),
            pl.BlockSpec((BN * K, 8), big),
            *[pl.BlockSpec(s, rep) for s in cshapes],
            pl.BlockSpec((8, H), rep),
            pl.BlockSpec((8, H), rep),
            pl.BlockSpec((1, H), rep),
            pl.BlockSpec((H, H), rep),
            pl.BlockSpec((1, H), rep),
            pl.BlockSpec((H, OUT), rep),
            pl.BlockSpec((1, OUT), rep),
        ],
        out_specs=pl.BlockSpec((BN, OUT), big),
        out_shape=jax.ShapeDtypeStruct((NP, OUT), jnp.float32),
    )(xg, pnj, pni, *consts, Ws8, Wc8, b1, W2, b2, W3, b3)


_NW = 32
_NQW = NP // _NW
_CM = 256
_CROW = _CM + 16
_R2BITS = np.float32(R * R).view(np.int32).item()
_SV = 1


def _stage(px_h, py_h, pz_h, gs_h, ge_h, pxv, pyv, pzv, gsv, gev, qbase):
    pltpu.sync_copy(px_h, pxv.at[pl.ds(0, NP)])
    pltpu.sync_copy(py_h, pyv.at[pl.ds(0, NP)])
    pltpu.sync_copy(pz_h, pzv.at[pl.ds(0, NP)])
    pltpu.sync_copy(gs_h.at[pl.ds(qbase, _NQW)], gsv.at[pl.ds(0, _NQW)])
    pltpu.sync_copy(ge_h.at[pl.ds(qbase, _NQW)], gev.at[pl.ds(0, _NQW)])


_GMAX = 2048


def _scan_body(px_h, py_h, pz_h, gs_h, ge_h, du_h,
               pxv, pyv, pzv, gsv, gev, rowb):
    i32 = jnp.int32
    wid = lax.axis_index("s") * 2 + lax.axis_index("c")
    qbase = wid * _NQW
    _stage(px_h, py_h, pz_h, gs_h, ge_h, pxv, pyv, pzv, gsv, gev, qbase)

    def per_query(qi, _):
        q = qbase + qi
        s = gsv[pl.ds(qi, 16)][0]
        e = gev[pl.ds(qi, 16)][0]
        qx = pxv[pl.ds(q, 16)][0]
        qy = pyv[pl.ds(q, 16)][0]
        qz = pzv[pl.ds(q, 16)][0]
        nchunks = (e - s + 15) // 16

        def scan_chunk(ci, _):
            base = s + ci * 16
            dx = pxv[pl.ds(base, 16)] - qx
            dy = pyv[pl.ds(base, 16)] - qy
            dz = pzv[pl.ds(base, 16)] - qz
            d2 = (dx * dx + dy * dy) + dz * dz
            rowb[pl.ds(ci * 16, 16)] = d2
            return 0

        lax.fori_loop(0, nchunks, scan_chunk, 0)
        pltpu.sync_copy(rowb.at[pl.ds(0, _GMAX)],
                        du_h.at[pl.ds(q * _GMAX, _GMAX)])
        return 0

    lax.fori_loop(0, _NQW, per_query, 0)


def _emitg_body(du_ref, gl_ref, gsx_ref, tg_ref, o_ref):
    i32 = jnp.int32
    f32 = jnp.float32
    bits = lax.bitcast_convert_type(du_ref[...], i32)
    glen = gl_ref[...]
    lanei = lax.broadcasted_iota(i32, (128, _GMAX), 1)
    valid = lanei < glen
    bits = jnp.where(valid, bits, jnp.int32(2147483647))

    def bs_body(_, lh):
        lo, hi = lh
        mid = lo + (hi - lo) // 2
        c = jnp.sum((bits <= mid).astype(i32), axis=1, keepdims=True)
        big = c >= K
        return jnp.where(big, lo, mid), jnp.where(big, mid, hi)

    lo0 = jnp.full((128, 1), -1, i32)
    hi0 = jnp.full((128, 1), _R2BITS, i32)
    _, tstar = lax.fori_loop(0, 31, bs_body, (lo0, hi0))

    sel = (bits <= tstar).astype(f32)
    rank = jnp.dot(sel.astype(jnp.bfloat16), tg_ref[...],
                   preferred_element_type=f32)
    cnt = jnp.minimum(jnp.sum(sel, axis=1, keepdims=True), jnp.float32(K))
    gsx = gsx_ref[...].astype(f32)
    idxval = (gsx + lanei.astype(f32)) * sel
    cols = []
    for j in range(K):
        pick = jnp.where(rank == jnp.float32(j + 1), idxval, 0.0)
        cols.append(jnp.sum(pick, axis=1, keepdims=True))
    V = jnp.concatenate(cols, axis=1)
    i = pl.program_id(0)
    qf = (i * 128 + lax.broadcasted_iota(i32, (128, K), 0)).astype(f32)
    colj = lax.broadcasted_iota(i32, (128, K), 1).astype(f32)
    o_ref[...] = jnp.where(colj < cnt, V, qf).astype(i32)


def _emit_call(du, glen, gsx):
    tg = jnp.asarray(np.triu(np.ones((_GMAX, _GMAX), np.float32))
                     .astype(jnp.bfloat16))
    return pl.pallas_call(
        _emitg_body,
        grid=(NP // 128,),
        in_specs=[pl.BlockSpec((128, _GMAX), lambda i: (i, 0)),
                  pl.BlockSpec((128, 1), lambda i: (i, 0)),
                  pl.BlockSpec((128, 1), lambda i: (i, 0)),
                  pl.BlockSpec((_GMAX, _GMAX), lambda i: (0, 0))],
        out_specs=pl.BlockSpec((128, K), lambda i: (i, 0)),
        out_shape=jax.ShapeDtypeStruct((NP, K), jnp.int32),
    )(du, glen, gsx, tg)


_GC = 512


def _gather_body(tbl_h, pn_h, idx_h, xg_h, pnj_h, idxv, rowsv, pnv, sem):
    cid = lax.axis_index("c")
    sid = lax.axis_index("s")
    wid = sid * 2 + cid
    nw = (NP * K) // _NW
    base = wid * nw

    def step(ci, _):
        off = base + ci * _GC
        pltpu.sync_copy(idx_h.at[pl.ds(off, _GC)], idxv)
        cp1 = pltpu.async_copy(tbl_h.at[idxv], rowsv, sem)
        cp2 = pltpu.async_copy(pn_h.at[idxv], pnv, sem)
        cp1.wait()
        cp2.wait()
        pltpu.sync_copy(rowsv, xg_h.at[pl.ds(off, _GC)])
        pltpu.sync_copy(pnv, pnj_h.at[pl.ds(off, _GC)])
        return 0

    lax.fori_loop(0, nw // _GC, step, 0)


def _sc_gather(tbl, pn, idx2d):
    mesh = plsc.VectorSubcoreMesh(core_axis_name="c", subcore_axis_name="s")
    f = pl.kernel(
        _gather_body, mesh=mesh,
        compiler_params=pltpu.CompilerParams(use_tc_tiling_on_sc=False),
        out_type=(jax.ShapeDtypeStruct((NP * K, H), jnp.float32),
                  jax.ShapeDtypeStruct((NP * K, 16), jnp.float32)),
        scratch_types=[
            pltpu.VMEM((_GC,), jnp.int32),
            pltpu.VMEM((_GC, H), jnp.float32),
            pltpu.VMEM((_GC, 16), jnp.float32),
            pltpu.SemaphoreType.DMA,
        ])
    return f(tbl, pn, idx2d)


def _sc_search_gather(px, py, pz, gs, ge, tbl, pn):
    mesh = plsc.VectorSubcoreMesh(core_axis_name="c", subcore_axis_name="s")
    stage_scratch = [
        pltpu.VMEM((NP + 16,), jnp.float32),
        pltpu.VMEM((NP + 16,), jnp.float32),
        pltpu.VMEM((NP + 16,), jnp.float32),
        pltpu.VMEM((_NQW + 16,), jnp.int32),
        pltpu.VMEM((_NQW + 16,), jnp.int32),
    ]
    scan = pl.kernel(
        _scan_body, mesh=mesh,
        out_type=jax.ShapeDtypeStruct((NP * _GMAX,), jnp.float32),
        scratch_types=stage_scratch + [
            pltpu.VMEM((_GMAX + 16,), jnp.float32),
        ])
    du = scan(px, py, pz, gs, ge)
    glen = (ge - gs).reshape(NP, 1)
    idx2d = _emit_call(du.reshape(NP, _GMAX), glen, gs.reshape(NP, 1))
    return _sc_gather(tbl, pn, idx2d.reshape(NP * K))


def _nbrs_jnp(pos, batch):
    chunks = []
    step = 2000
    for s in range(0, pos.shape[0], step):
        q = pos[s:s + step]
        d2 = jnp.sum((q[:, None, :] - pos[None, :, :]) ** 2, axis=-1)
        valid = (batch[s:s + step][:, None] == batch[None, :]) & (d2 <= R * R)
        d2m = jnp.where(valid, d2, 1e30)
        vals, idx = lax.top_k(-d2m, K)
        qi = jnp.arange(s, s + q.shape[0], dtype=idx.dtype)[:, None]
        idx = jnp.where(vals <= -1e29, qi, idx)
        chunks.append(idx)
    return jnp.concatenate(chunks, axis=0)


def kernel(x, pos, normal, batch, W1, b1, W2, b2, W3, b3):
    W1a = W1[:D]
    W1b = W1[D:]
    zrow = jnp.zeros((1, H), jnp.float32)
    Ws8 = jnp.concatenate(
        [W1b[0:1], W1b[1:2], W1b[3:4], W1b[5:6], zrow, zrow, zrow, zrow], 0)
    Wc8 = jnp.concatenate(
        [zrow, W1b[2:3], W1b[4:5], W1b[6:7], zrow, zrow, zrow, zrow], 0)
    x_p = jnp.pad(x, ((0, NP - N), (0, 0)))
    xw = _xw_matmul(x_p, W1a)

    pos_p = jnp.pad(pos, ((0, NP - N), (0, 0)))
    px, py, pz = pos_p[:, 0], pos_p[:, 1], pos_p[:, 2]
    starts = jnp.searchsorted(batch, jnp.arange(NB, dtype=batch.dtype),
                              side="left").astype(jnp.int32)
    ends = jnp.searchsorted(batch, jnp.arange(NB, dtype=batch.dtype),
                            side="right").astype(jnp.int32)
    gs = jnp.pad(starts[batch], (0, NP - N))
    ge = jnp.pad(ends[batch], (0, NP - N))
    pn = jnp.concatenate(
        [pos, normal, jnp.zeros((N, 10), jnp.float32)], axis=1)
    pn_p = jnp.pad(pn, ((0, NP - N), (0, 0)))

    xg, pnj = _sc_search_gather(px, py, pz, gs, ge, xw, pn_p)
    pni = jnp.repeat(pn_p[:, :8], K, axis=0)

    b1r = b1.reshape(1, H)
    b2r = b2.reshape(1, H)
    b3r = b3.reshape(1, OUT)
    out_full = _mlp_call(xg, pnj, pni, Ws8, Wc8, b1r, W2, b2r, W3, b3r)
    return (out_full[:N], pos, batch)


# speedup vs baseline: 16.9409x; 1.0620x over previous
import functools

import jax
import jax.numpy as jnp
import numpy as np
from jax import lax
from jax.experimental import pallas as pl
from jax.experimental.pallas import tpu as pltpu
from jax.experimental.pallas import tpu_sc as plsc

N = 10000
D = 128
K = 64
R = 0.25
NB = 8
H = 64
OUT = 128
NP = 10240
BN = 128


def _mm_body(x_ref, w_ref, o_ref):
    o_ref[...] = jnp.dot(x_ref[...], w_ref[...],
                         preferred_element_type=jnp.float32)


def _xw_matmul(x_p, W1a):
    return pl.pallas_call(
        _mm_body,
        grid=(NP // 128,),
        in_specs=[pl.BlockSpec((128, D), lambda i: (i, 0)),
                  pl.BlockSpec((D, H), lambda i: (0, 0))],
        out_specs=pl.BlockSpec((128, H), lambda i: (i, 0)),
        out_shape=jax.ShapeDtypeStruct((NP, H), jnp.float32),
    )(x_p, W1a)


_PAIRS = [
    (0, 0), (1, 1), (2, 2),
    (3, 0), (4, 1), (5, 2),
    (6, 0), (7, 1), (8, 2),
    (3, 6), (4, 7), (5, 8),
    (4, 2), (5, 1),
    (5, 0), (3, 2),
    (3, 1), (4, 0),
    (7, 2), (8, 1),
    (8, 0), (6, 2),
    (6, 1), (7, 0),
    (4, 8), (5, 7),
    (5, 6), (3, 8),
    (3, 7), (4, 6),
]


def _make_consts():
    M1 = np.zeros((16, 16), np.float32)
    M2 = np.zeros((8, 16), np.float32)
    for a in range(3):
        M1[a, a] = 1.0
        M2[a, a] = -1.0
        M1[3 + a, 6 + a] = 1.0
        M2[3 + a, 3 + a] = 1.0
    A = np.zeros((16, 32), np.float32)
    B = np.zeros((16, 32), np.float32)
    for j, (a, b) in enumerate(_PAIRS):
        A[a, j] = 1.0
        B[b, j] = 1.0
    C1 = np.zeros((32, 16), np.float32)
    for j in range(3):
        C1[j, 0] = 1.0
        C1[3 + j, 1] = 1.0
        C1[6 + j, 2] = 1.0
        C1[9 + j, 3] = 1.0
    for c in range(9):
        C1[12 + 2 * c, 4 + c] = 1.0
        C1[13 + 2 * c, 4 + c] = -1.0
    C2 = np.zeros((16, 8), np.float32)
    for k in range(3):
        for c in range(3):
            C2[4 + 3 * k + c, 1 + k] = 1.0
    return (jnp.asarray(M1), jnp.asarray(M2), jnp.asarray(A),
            jnp.asarray(B), jnp.asarray(C1), jnp.asarray(C2))


def _mlp_body(xg_ref, pnj_ref, pni_ref, m1_ref, m2_ref, a_ref, b_ref,
              c1_ref, c2_ref, ws_ref, wc_ref, b1_ref, w2_ref, b2_ref,
              w3_ref, b3_ref, o_ref):
    f32 = jnp.float32

    def mm(a, b):
        return jnp.dot(a, b, preferred_element_type=f32)

    pnj = pnj_ref[...]
    pni = pni_ref[...]
    Z = mm(pnj, m1_ref[...]) + mm(pni, m2_ref[...])
    P1 = mm(Z, a_ref[...]) * mm(Z, b_ref[...])
    P2 = mm(P1, c1_ref[...])
    P3 = P2 * P2
    P2s = P2[:, 0:8]
    lane = lax.broadcasted_iota(jnp.int32, P2s.shape, 1)
    Cv = jnp.where(lane == 0, P2s, mm(P3, c2_ref[...]))
    Nv = jnp.sqrt(Cv)
    H2 = Nv * Nv + P2s * P2s
    Rv = lax.rsqrt(jnp.where(H2 == 0.0, 1.0, H2))
    SINf = jnp.where(lane == 0, Nv * (1.0 / R), Nv * Rv)
    COSv = jnp.where(H2 == 0.0, 1.0, P2s * Rv)
    h1ppf = mm(SINf, ws_ref[...]) + mm(COSv, wc_ref[...])
    h1 = jnp.maximum(xg_ref[...] + h1ppf + b1_ref[...], 0.0)
    h2 = mm(h1, w2_ref[...]) + b2_ref[...]
    hm = jnp.max(h2.reshape(BN, K, H), axis=1)
    o_ref[...] = mm(hm, w3_ref[...]) + b3_ref[...]


def _mlp_call(xg, pnj, pni, Ws8, Wc8, b1, W2, b2, W3, b3):
    nblk = NP // BN
    consts = _make_consts()
    big = lambda i: (i, 0)
    rep = lambda i: (0, 0)
    cshapes = [(16, 16), (8, 16), (16, 32), (16, 32), (32, 16), (16, 8)]
    return pl.pallas_call(
        _mlp_body,
        grid=(nblk,),
        in_specs=[
            pl.BlockSpec((BN * K, H), big),
            pl.BlockSpec((BN * K, 16), big),
            pl.BlockSpec((BN * K, 8), big),
            *[pl.BlockSpec(s, rep) for s in cshapes],
            pl.BlockSpec((8, H), rep),
            pl.BlockSpec((8, H), rep),
            pl.BlockSpec((1, H), rep),
            pl.BlockSpec((H, H), rep),
            pl.BlockSpec((1, H), rep),
            pl.BlockSpec((H, OUT), rep),
            pl.BlockSpec((1, OUT), rep),
        ],
        out_specs=pl.BlockSpec((BN, OUT), big),
        out_shape=jax.ShapeDtypeStruct((NP, OUT), jnp.float32),
    )(xg, pnj, pni, *consts, Ws8, Wc8, b1, W2, b2, W3, b3)


_NW = 32
_NQW = NP // _NW
_CM = 256
_CROW = _CM + 16
_R2BITS = np.float32(R * R).view(np.int32).item()
_SV = 1


def _stage(px_h, py_h, pz_h, gs_h, ge_h, pxv, pyv, pzv, gsv, gev, qbase):
    pltpu.sync_copy(px_h, pxv.at[pl.ds(0, NP)])
    pltpu.sync_copy(py_h, pyv.at[pl.ds(0, NP)])
    pltpu.sync_copy(pz_h, pzv.at[pl.ds(0, NP)])
    pltpu.sync_copy(gs_h.at[pl.ds(qbase, _NQW)], gsv.at[pl.ds(0, _NQW)])
    pltpu.sync_copy(ge_h.at[pl.ds(qbase, _NQW)], gev.at[pl.ds(0, _NQW)])


_GMAX = 1664


def _scan_body(px_h, py_h, pz_h, gs_h, ge_h, du_h,
               pxv, pyv, pzv, gsv, gev, rowb):
    i32 = jnp.int32
    wid = lax.axis_index("s") * 2 + lax.axis_index("c")
    qbase = wid * _NQW
    _stage(px_h, py_h, pz_h, gs_h, ge_h, pxv, pyv, pzv, gsv, gev, qbase)

    def per_query(qi, _):
        q = qbase + qi
        s = gsv[pl.ds(qi, 16)][0]
        e = gev[pl.ds(qi, 16)][0]
        qx = pxv[pl.ds(q, 16)][0]
        qy = pyv[pl.ds(q, 16)][0]
        qz = pzv[pl.ds(q, 16)][0]
        nchunks = (e - s + 15) // 16

        def scan_chunk(ci, _):
            base = s + ci * 16
            dx = pxv[pl.ds(base, 16)] - qx
            dy = pyv[pl.ds(base, 16)] - qy
            dz = pzv[pl.ds(base, 16)] - qz
            d2 = (dx * dx + dy * dy) + dz * dz
            rowb[pl.ds(ci * 16, 16)] = d2
            return 0

        lax.fori_loop(0, nchunks, scan_chunk, 0)
        pltpu.sync_copy(rowb.at[pl.ds(0, _GMAX)],
                        du_h.at[pl.ds(q * _GMAX, _GMAX)])
        return 0

    lax.fori_loop(0, _NQW, per_query, 0)


def _emitg_body(du_ref, gl_ref, gsx_ref, tg_ref, o_ref):
    i32 = jnp.int32
    f32 = jnp.float32
    bits = lax.bitcast_convert_type(du_ref[...], i32)
    glen = gl_ref[...]
    lanei = lax.broadcasted_iota(i32, (128, _GMAX), 1)
    valid = lanei < glen
    bits = jnp.where(valid, bits, jnp.int32(2147483647))

    def bs_body(_, lh):
        lo, hi = lh
        mid = lo + (hi - lo) // 2
        c = jnp.sum((bits <= mid).astype(i32), axis=1, keepdims=True)
        big = c >= K
        return jnp.where(big, lo, mid), jnp.where(big, mid, hi)

    lo0 = jnp.full((128, 1), -1, i32)
    hi0 = jnp.full((128, 1), _R2BITS, i32)
    _, tstar = lax.fori_loop(0, 31, bs_body, (lo0, hi0))

    sel = (bits <= tstar).astype(f32)
    rank = jnp.dot(sel.astype(jnp.bfloat16), tg_ref[...],
                   preferred_element_type=f32)
    cnt = jnp.minimum(jnp.sum(sel, axis=1, keepdims=True), jnp.float32(K))
    gsx = gsx_ref[...].astype(f32)
    idxval = (gsx + lanei.astype(f32)) * sel
    cols = []
    for j in range(K):
        pick = jnp.where(rank == jnp.float32(j + 1), idxval, 0.0)
        cols.append(jnp.sum(pick, axis=1, keepdims=True))
    V = jnp.concatenate(cols, axis=1)
    i = pl.program_id(0)
    qf = (i * 128 + lax.broadcasted_iota(i32, (128, K), 0)).astype(f32)
    colj = lax.broadcasted_iota(i32, (128, K), 1).astype(f32)
    o_ref[...] = jnp.where(colj < cnt, V, qf).astype(i32)


def _emit_call(du, glen, gsx):
    tg = jnp.asarray(np.triu(np.ones((_GMAX, _GMAX), np.float32))
                     .astype(jnp.bfloat16))
    return pl.pallas_call(
        _emitg_body,
        grid=(NP // 128,),
        in_specs=[pl.BlockSpec((128, _GMAX), lambda i: (i, 0)),
                  pl.BlockSpec((128, 1), lambda i: (i, 0)),
                  pl.BlockSpec((128, 1), lambda i: (i, 0)),
                  pl.BlockSpec((_GMAX, _GMAX), lambda i: (0, 0))],
        out_specs=pl.BlockSpec((128, K), lambda i: (i, 0)),
        out_shape=jax.ShapeDtypeStruct((NP, K), jnp.int32),
    )(du, glen, gsx, tg)


_GC = 512


def _gather_body(tbl_h, pn_h, idx_h, xg_h, pnj_h, idxv, rowsv, pnv, sem):
    cid = lax.axis_index("c")
    sid = lax.axis_index("s")
    wid = sid * 2 + cid
    nw = (NP * K) // _NW
    base = wid * nw

    def step(ci, _):
        off = base + ci * _GC
        pltpu.sync_copy(idx_h.at[pl.ds(off, _GC)], idxv)
        cp1 = pltpu.async_copy(tbl_h.at[idxv], rowsv, sem)
        cp2 = pltpu.async_copy(pn_h.at[idxv], pnv, sem)
        cp1.wait()
        cp2.wait()
        pltpu.sync_copy(rowsv, xg_h.at[pl.ds(off, _GC)])
        pltpu.sync_copy(pnv, pnj_h.at[pl.ds(off, _GC)])
        return 0

    lax.fori_loop(0, nw // _GC, step, 0)


def _sc_gather(tbl, pn, idx2d):
    mesh = plsc.VectorSubcoreMesh(core_axis_name="c", subcore_axis_name="s")
    f = pl.kernel(
        _gather_body, mesh=mesh,
        compiler_params=pltpu.CompilerParams(use_tc_tiling_on_sc=False),
        out_type=(jax.ShapeDtypeStruct((NP * K, H), jnp.float32),
                  jax.ShapeDtypeStruct((NP * K, 16), jnp.float32)),
        scratch_types=[
            pltpu.VMEM((_GC,), jnp.int32),
            pltpu.VMEM((_GC, H), jnp.float32),
            pltpu.VMEM((_GC, 16), jnp.float32),
            pltpu.SemaphoreType.DMA,
        ])
    return f(tbl, pn, idx2d)


def _sc_search_gather(px, py, pz, gs, ge, tbl, pn):
    mesh = plsc.VectorSubcoreMesh(core_axis_name="c", subcore_axis_name="s")
    stage_scratch = [
        pltpu.VMEM((NP + 16,), jnp.float32),
        pltpu.VMEM((NP + 16,), jnp.float32),
        pltpu.VMEM((NP + 16,), jnp.float32),
        pltpu.VMEM((_NQW + 16,), jnp.int32),
        pltpu.VMEM((_NQW + 16,), jnp.int32),
    ]
    scan = pl.kernel(
        _scan_body, mesh=mesh,
        out_type=jax.ShapeDtypeStruct((NP * _GMAX,), jnp.float32),
        scratch_types=stage_scratch + [
            pltpu.VMEM((_GMAX + 16,), jnp.float32),
        ])
    du = scan(px, py, pz, gs, ge)
    glen = (ge - gs).reshape(NP, 1)
    idx2d = _emit_call(du.reshape(NP, _GMAX), glen, gs.reshape(NP, 1))
    return _sc_gather(tbl, pn, idx2d.reshape(NP * K))


def _nbrs_jnp(pos, batch):
    chunks = []
    step = 2000
    for s in range(0, pos.shape[0], step):
        q = pos[s:s + step]
        d2 = jnp.sum((q[:, None, :] - pos[None, :, :]) ** 2, axis=-1)
        valid = (batch[s:s + step][:, None] == batch[None, :]) & (d2 <= R * R)
        d2m = jnp.where(valid, d2, 1e30)
        vals, idx = lax.top_k(-d2m, K)
        qi = jnp.arange(s, s + q.shape[0], dtype=idx.dtype)[:, None]
        idx = jnp.where(vals <= -1e29, qi, idx)
        chunks.append(idx)
    return jnp.concatenate(chunks, axis=0)


def kernel(x, pos, normal, batch, W1, b1, W2, b2, W3, b3):
    W1a = W1[:D]
    W1b = W1[D:]
    zrow = jnp.zeros((1, H), jnp.float32)
    Ws8 = jnp.concatenate(
        [W1b[0:1], W1b[1:2], W1b[3:4], W1b[5:6], zrow, zrow, zrow, zrow], 0)
    Wc8 = jnp.concatenate(
        [zrow, W1b[2:3], W1b[4:5], W1b[6:7], zrow, zrow, zrow, zrow], 0)
    x_p = jnp.pad(x, ((0, NP - N), (0, 0)))
    xw = _xw_matmul(x_p, W1a)

    pos_p = jnp.pad(pos, ((0, NP - N), (0, 0)))
    px, py, pz = pos_p[:, 0], pos_p[:, 1], pos_p[:, 2]
    starts = jnp.searchsorted(batch, jnp.arange(NB, dtype=batch.dtype),
                              side="left").astype(jnp.int32)
    ends = jnp.searchsorted(batch, jnp.arange(NB, dtype=batch.dtype),
                            side="right").astype(jnp.int32)
    gs = jnp.pad(starts[batch], (0, NP - N))
    ge = jnp.pad(ends[batch], (0, NP - N))
    pn = jnp.concatenate(
        [pos, normal, jnp.zeros((N, 10), jnp.float32)], axis=1)
    pn_p = jnp.pad(pn, ((0, NP - N), (0, 0)))

    xg, pnj = _sc_search_gather(px, py, pz, gs, ge, xw, pn_p)
    pni = jnp.repeat(pn_p[:, :8], K, axis=0)

    b1r = b1.reshape(1, H)
    b2r = b2.reshape(1, H)
    b3r = b3.reshape(1, OUT)
    out_full = _mlp_call(xg, pnj, pni, Ws8, Wc8, b1r, W2, b2r, W3, b3r)
    return (out_full[:N], pos, batch)
